# baseline (speedup 1.0000x reference)
import jax
import jax.numpy as jnp
from jax.experimental import pallas as pl
from jax.experimental.pallas import tpu as pltpu
from jax.experimental.pallas import tpu_sc as plsc

_ROWS = 16
_LANES = 16


def kernel(inputs, pos_table):
    batch, seq, emb = inputs.shape
    x = inputs.reshape(batch * seq, emb)
    nseq = seq // _ROWS
    mesh = plsc.VectorSubcoreMesh(core_axis_name="c", subcore_axis_name="s")

    @pl.kernel(
        out_type=jax.ShapeDtypeStruct((batch * seq, emb), inputs.dtype),
        mesh=mesh,
    )
    def sc_kernel(x_hbm, p_hbm, o_hbm):
        def body(x_vmem, p_vmem, o_vmem):
            @pl.loop(0, _ROWS)
            def _(r):
                for c in range(0, emb, _LANES):
                    slc = (pl.ds(r, 1), pl.ds(c, _LANES))
                    o_vmem.at[*slc][...] = (
                        x_vmem.at[*slc][...] + p_vmem.at[*slc][...]
                    )

        pltpu.emit_pipeline(
            body,
            grid=(nseq, batch),
            in_specs=[
                pl.BlockSpec((_ROWS, emb), index_map=lambda i, j: (j * nseq + i, 0)),
                pl.BlockSpec((_ROWS, emb), index_map=lambda i, j: (i, 0)),
            ],
            out_specs=[
                pl.BlockSpec((_ROWS, emb), index_map=lambda i, j: (j * nseq + i, 0)),
            ],
            core_axis_name=("c", "s"),
            dimension_semantics=(pltpu.PARALLEL, pltpu.ARBITRARY),
        )(x_hbm, p_hbm, o_hbm)

    out = sc_kernel(x, pos_table)
    return out.reshape(batch, seq, emb)


# speedup vs baseline: 3.8488x; 3.8488x over previous
import jax
import jax.numpy as jnp
from jax.experimental import pallas as pl
from jax.experimental.pallas import tpu as pltpu

_BLOCK_S = 256


def _add_kernel(in_ref, pos_ref, out_ref):
    out_ref[...] = in_ref[...] + pos_ref[...][None, :, :]


def kernel(inputs, pos_table):
    batch, seq, emb = inputs.shape
    bs = _BLOCK_S
    grid = (seq // bs,)
    return pl.pallas_call(
        _add_kernel,
        grid=grid,
        in_specs=[
            pl.BlockSpec((batch, bs, emb), lambda s: (0, s, 0)),
            pl.BlockSpec((bs, emb), lambda s: (s, 0)),
        ],
        out_specs=pl.BlockSpec((batch, bs, emb), lambda s: (0, s, 0)),
        out_shape=jax.ShapeDtypeStruct(inputs.shape, inputs.dtype),
        compiler_params=pltpu.CompilerParams(
            dimension_semantics=("arbitrary",),
        ),
    )(inputs, pos_table)


# speedup vs baseline: 3.9626x; 1.0296x over previous
import jax
import jax.numpy as jnp
from jax.experimental import pallas as pl
from jax.experimental.pallas import tpu as pltpu

_BLOCK_S = 512


def _add_kernel(in_ref, pos_ref, out_ref):
    out_ref[...] = in_ref[...] + pos_ref[...][None, :, :]


def kernel(inputs, pos_table):
    batch, seq, emb = inputs.shape
    bs = _BLOCK_S
    grid = (seq // bs,)
    return pl.pallas_call(
        _add_kernel,
        grid=grid,
        in_specs=[
            pl.BlockSpec((batch, bs, emb), lambda s: (0, s, 0)),
            pl.BlockSpec((bs, emb), lambda s: (s, 0)),
        ],
        out_specs=pl.BlockSpec((batch, bs, emb), lambda s: (0, s, 0)),
        out_shape=jax.ShapeDtypeStruct(inputs.shape, inputs.dtype),
        compiler_params=pltpu.CompilerParams(
            dimension_semantics=("arbitrary",),
        ),
    )(inputs, pos_table)
